# Initial kernel scaffold; baseline (speedup 1.0000x reference)
import jax
import jax.numpy as jnp
from jax.experimental import pallas as pl


def kernel(input, Nu, uspan):
    raise NotImplementedError("write your pallas kernel here")


# speedup vs baseline: 1.0232x; 1.0232x over previous
import dataclasses
import functools

import jax
import jax.numpy as jnp
from jax import lax
from jax.experimental import pallas as pl
from jax.experimental.pallas import tpu as pltpu
from jax.experimental.pallas import tpu_sc as plsc

B = 1024
M = 1024
P = 2
DIM = 3
OUT_DIM = 2048

NUM_CORES = 2
NUM_SUBCORES = 16
LANES = 16
NUM_TILES = NUM_CORES * NUM_SUBCORES
CURVES_PER_TILE = B // NUM_TILES
NUM_CHUNKS = OUT_DIM // LANES


def _sc_curve_eval(inp_flat, nut, ibase4):
  mesh = plsc.VectorSubcoreMesh(core_axis_name="c", subcore_axis_name="s")
  cp = pltpu.CompilerParams()
  if "needs_layout_passes" in pltpu.CompilerParams.__dataclass_fields__:
    cp = dataclasses.replace(cp, needs_layout_passes=False)

  @functools.partial(
      pl.kernel,
      compiler_params=cp,
      out_type=jax.ShapeDtypeStruct((B, OUT_DIM * DIM), jnp.float32),
      mesh=mesh,
      scratch_types=[
          pltpu.VMEM((M * 4,), jnp.float32),
          pltpu.VMEM((OUT_DIM * DIM,), jnp.float32),
          pltpu.VMEM((P + 1, OUT_DIM), jnp.float32),
          pltpu.VMEM((OUT_DIM,), jnp.int32),
      ],
  )
  def k(inp_hbm, nut_hbm, idx_hbm, out_hbm, cp_v, out_v, nut_v, idx_v):
    wid = lax.axis_index("s") * NUM_CORES + lax.axis_index("c")
    pltpu.sync_copy(nut_hbm, nut_v)
    pltpu.sync_copy(idx_hbm, idx_v)
    lane = lax.iota(jnp.int32, LANES)
    lane3 = lane * 3

    @pl.loop(0, CURVES_PER_TILE)
    def _(i):
      b = wid * CURVES_PER_TILE + i
      pltpu.sync_copy(inp_hbm.at[b], cp_v)

      @pl.loop(0, NUM_CHUNKS)
      def _(kk):
        t0 = kk * LANES
        ibase = idx_v[pl.ds(t0, LANES)]
        nu0 = nut_v[0, pl.ds(t0, LANES)]
        nu1 = nut_v[1, pl.ds(t0, LANES)]
        nu2 = nut_v[2, pl.ds(t0, LANES)]
        acc = [None] * 4
        for c in range(4):
          g0 = plsc.load_gather(cp_v, [ibase + c])
          g1 = plsc.load_gather(cp_v, [ibase + (4 + c)])
          g2 = plsc.load_gather(cp_v, [ibase + (8 + c)])
          acc[c] = nu0 * g0 + nu1 * g1 + nu2 * g2
        winv = 1.0 / acc[3]
        obase = lane3 + t0 * 3
        plsc.store_scatter(out_v, [obase], acc[0] * winv)
        plsc.store_scatter(out_v, [obase + 1], acc[1] * winv)
        plsc.store_scatter(out_v, [obase + 2], acc[2] * winv)

      pltpu.sync_copy(out_v, out_hbm.at[b])

  return k(inp_flat, nut, ibase4)


def kernel(input, Nu, uspan):
  inp_flat = input.reshape(B, M * 4)
  nut = Nu.T
  ibase4 = (uspan - P) * 4
  out = _sc_curve_eval(inp_flat, nut, ibase4)
  return out.reshape(B, OUT_DIM, DIM)


# speedup vs baseline: 1.1701x; 1.1435x over previous
import dataclasses
import functools

import jax
import jax.numpy as jnp
from jax import lax
from jax.experimental import pallas as pl
from jax.experimental.pallas import tpu as pltpu
from jax.experimental.pallas import tpu_sc as plsc

B = 1024
M = 1024
P = 2
DIM = 3
OUT_DIM = 2048

NUM_CORES = 2
NUM_SUBCORES = 16
LANES = 16
NUM_TILES = NUM_CORES * NUM_SUBCORES
CURVES_PER_TILE = B // NUM_TILES
NUM_CHUNKS = OUT_DIM // LANES


def _sc_curve_eval(inp_flat, nut, ibase4):
  mesh = plsc.VectorSubcoreMesh(core_axis_name="c", subcore_axis_name="s")
  cp = pltpu.CompilerParams()
  if "needs_layout_passes" in pltpu.CompilerParams.__dataclass_fields__:
    cp = dataclasses.replace(cp, needs_layout_passes=False)

  npairs = CURVES_PER_TILE // 2

  @functools.partial(
      pl.kernel,
      compiler_params=cp,
      out_type=jax.ShapeDtypeStruct((B, OUT_DIM * DIM), jnp.float32),
      mesh=mesh,
      scratch_types=[
          pltpu.VMEM((M * 4,), jnp.float32),
          pltpu.VMEM((M * 4,), jnp.float32),
          pltpu.VMEM((OUT_DIM * DIM,), jnp.float32),
          pltpu.VMEM((OUT_DIM * DIM,), jnp.float32),
          pltpu.VMEM((P + 1, OUT_DIM), jnp.float32),
          pltpu.VMEM((OUT_DIM,), jnp.int32),
          pltpu.SemaphoreType.DMA,
          pltpu.SemaphoreType.DMA,
          pltpu.SemaphoreType.DMA,
          pltpu.SemaphoreType.DMA,
          pltpu.SemaphoreType.DMA,
      ],
  )
  def k(inp_hbm, nut_hbm, idx_hbm, out_hbm, cp0, cp1, ob0, ob1, nut_v, idx_v,
        isem0, isem1, osem0, osem1, csem):
    wid = lax.axis_index("s") * NUM_CORES + lax.axis_index("c")
    base = wid * CURVES_PER_TILE
    pltpu.async_copy(nut_hbm, nut_v, csem)
    pltpu.async_copy(idx_hbm, idx_v, csem)
    pltpu.async_copy(inp_hbm.at[base], cp0, isem0)
    pltpu.make_async_copy(nut_hbm, nut_v, csem).wait()
    pltpu.make_async_copy(idx_hbm, idx_v, csem).wait()
    lane3 = lax.iota(jnp.int32, LANES) * 3

    def compute(cpb, outb):
      @pl.loop(0, NUM_CHUNKS)
      def _(kk):
        t0 = kk * LANES
        ibase = idx_v[pl.ds(t0, LANES)]
        nu0 = nut_v[0, pl.ds(t0, LANES)]
        nu1 = nut_v[1, pl.ds(t0, LANES)]
        nu2 = nut_v[2, pl.ds(t0, LANES)]
        acc = [None] * 4
        for c in range(4):
          g0 = plsc.load_gather(cpb, [ibase + c])
          g1 = plsc.load_gather(cpb, [ibase + (4 + c)])
          g2 = plsc.load_gather(cpb, [ibase + (8 + c)])
          acc[c] = nu0 * g0 + nu1 * g1 + nu2 * g2
        winv = 1.0 / acc[3]
        obase = lane3 + t0 * 3
        plsc.store_scatter(outb, [obase], acc[0] * winv)
        plsc.store_scatter(outb, [obase + 1], acc[1] * winv)
        plsc.store_scatter(outb, [obase + 2], acc[2] * winv)

    @pl.loop(0, npairs)
    def _(ii):
      b0 = base + 2 * ii
      b1 = b0 + 1
      pltpu.make_async_copy(inp_hbm.at[b0], cp0, isem0).wait()
      pltpu.async_copy(inp_hbm.at[b1], cp1, isem1)

      @pl.when(ii > 0)
      def _():
        pltpu.make_async_copy(ob0, out_hbm.at[b0], osem0).wait()

      compute(cp0, ob0)
      pltpu.async_copy(ob0, out_hbm.at[b0], osem0)

      pltpu.make_async_copy(inp_hbm.at[b1], cp1, isem1).wait()

      @pl.when(ii < npairs - 1)
      def _():
        pltpu.async_copy(inp_hbm.at[b0 + 2], cp0, isem0)

      @pl.when(ii > 0)
      def _():
        pltpu.make_async_copy(ob1, out_hbm.at[b1], osem1).wait()

      compute(cp1, ob1)
      pltpu.async_copy(ob1, out_hbm.at[b1], osem1)

    pltpu.make_async_copy(ob0, out_hbm.at[base], osem0).wait()
    pltpu.make_async_copy(ob1, out_hbm.at[base], osem1).wait()

  return k(inp_flat, nut, ibase4)


def kernel(input, Nu, uspan):
  inp_flat = input.reshape(B, M * 4)
  nut = Nu.T
  ibase4 = (uspan - P) * 4
  out = _sc_curve_eval(inp_flat, nut, ibase4)
  return out.reshape(B, OUT_DIM, DIM)


# speedup vs baseline: 1.2616x; 1.0782x over previous
import dataclasses
import functools

import jax
import jax.numpy as jnp
from jax import lax
from jax.experimental import pallas as pl
from jax.experimental.pallas import tpu as pltpu
from jax.experimental.pallas import tpu_sc as plsc

B = 1024
M = 1024
P = 2
DIM = 3
OUT_DIM = 2048

NUM_CORES = 2
NUM_SUBCORES = 16
LANES = 16
NUM_TILES = NUM_CORES * NUM_SUBCORES
CURVES_PER_TILE = B // NUM_TILES
NUM_CHUNKS = OUT_DIM // LANES


def _sc_curve_eval(inp_flat, nut, ibase4):
  mesh = plsc.VectorSubcoreMesh(core_axis_name="c", subcore_axis_name="s")
  cp = pltpu.CompilerParams()
  if "needs_layout_passes" in pltpu.CompilerParams.__dataclass_fields__:
    cp = dataclasses.replace(cp, needs_layout_passes=False)

  G = 4
  ngroups = CURVES_PER_TILE // G
  npairs = ngroups // 2

  @functools.partial(
      pl.kernel,
      compiler_params=cp,
      out_type=jax.ShapeDtypeStruct((B, OUT_DIM * DIM), jnp.float32),
      mesh=mesh,
      scratch_types=(
          [pltpu.VMEM((M * 4,), jnp.float32)] * (2 * G)
          + [pltpu.VMEM((OUT_DIM * DIM,), jnp.float32)] * (2 * G)
          + [
              pltpu.VMEM((P + 1, OUT_DIM), jnp.float32),
              pltpu.VMEM((OUT_DIM,), jnp.int32),
              pltpu.SemaphoreType.DMA,
              pltpu.SemaphoreType.DMA,
              pltpu.SemaphoreType.DMA,
              pltpu.SemaphoreType.DMA,
              pltpu.SemaphoreType.DMA,
          ]
      ),
  )
  def k(inp_hbm, nut_hbm, idx_hbm, out_hbm, *scratch):
    cpbuf = [scratch[:G], scratch[G:2 * G]]
    obuf = [scratch[2 * G:3 * G], scratch[3 * G:4 * G]]
    nut_v, idx_v, isem0, isem1, osem0, osem1, csem = scratch[4 * G:]
    isem = [isem0, isem1]
    osem = [osem0, osem1]
    wid = lax.axis_index("s") * NUM_CORES + lax.axis_index("c")
    base = wid * CURVES_PER_TILE

    def start_in(buf, b0):
      for g in range(G):
        pltpu.async_copy(inp_hbm.at[b0 + g], cpbuf[buf][g], isem[buf])

    def wait_in(buf, b0):
      for g in range(G):
        pltpu.make_async_copy(inp_hbm.at[b0 + g], cpbuf[buf][g],
                              isem[buf]).wait()

    def start_out(buf, b0):
      for g in range(G):
        pltpu.async_copy(obuf[buf][g], out_hbm.at[b0 + g], osem[buf])

    def wait_out(buf, b0):
      for g in range(G):
        pltpu.make_async_copy(obuf[buf][g], out_hbm.at[b0 + g],
                              osem[buf]).wait()

    pltpu.async_copy(nut_hbm, nut_v, csem)
    pltpu.async_copy(idx_hbm, idx_v, csem)
    start_in(0, base)
    pltpu.make_async_copy(nut_hbm, nut_v, csem).wait()
    pltpu.make_async_copy(idx_hbm, idx_v, csem).wait()
    lane3 = lax.iota(jnp.int32, LANES) * 3

    def compute(buf):
      @pl.loop(0, NUM_CHUNKS)
      def _(kk):
        t0 = kk * LANES
        ibase = idx_v[pl.ds(t0, LANES)]
        nu0 = nut_v[0, pl.ds(t0, LANES)]
        nu1 = nut_v[1, pl.ds(t0, LANES)]
        nu2 = nut_v[2, pl.ds(t0, LANES)]
        gidx = [ibase + n for n in range(12)]
        obase = lane3 + t0 * 3
        for g in range(G):
          cpg = cpbuf[buf][g]
          acc = [None] * 4
          for c in range(4):
            g0 = plsc.load_gather(cpg, [gidx[c]])
            g1 = plsc.load_gather(cpg, [gidx[4 + c]])
            g2 = plsc.load_gather(cpg, [gidx[8 + c]])
            acc[c] = nu0 * g0 + nu1 * g1 + nu2 * g2
          winv = 1.0 / acc[3]
          og = obuf[buf][g]
          plsc.store_scatter(og, [obase], acc[0] * winv)
          plsc.store_scatter(og, [obase + 1], acc[1] * winv)
          plsc.store_scatter(og, [obase + 2], acc[2] * winv)

    @pl.loop(0, npairs)
    def _(ii):
      b0 = base + 2 * G * ii
      b1 = b0 + G
      wait_in(0, b0)
      start_in(1, b1)

      @pl.when(ii > 0)
      def _():
        wait_out(0, b0)

      compute(0)
      start_out(0, b0)

      wait_in(1, b1)

      @pl.when(ii < npairs - 1)
      def _():
        start_in(0, b0 + 2 * G)

      @pl.when(ii > 0)
      def _():
        wait_out(1, b1)

      compute(1)
      start_out(1, b1)

    wait_out(0, base)
    wait_out(1, base)

  return k(inp_flat, nut, ibase4)


def kernel(input, Nu, uspan):
  inp_flat = input.reshape(B, M * 4)
  nut = Nu.T
  ibase4 = (uspan - P) * 4
  out = _sc_curve_eval(inp_flat, nut, ibase4)
  return out.reshape(B, OUT_DIM, DIM)
